# baseline (speedup 1.0000x reference)
import jax
import jax.numpy as jnp
from jax.experimental import pallas as pl
from jax.experimental.pallas import tpu as pltpu

_TEMPERATURE = 0.07
_ROW_TILE = 256


def _loss_tile_kernel(emb_ref, tok_ref, loss_ref, pairs_ref):
    i = pl.program_id(0)
    r = _ROW_TILE
    emb = emb_ref[...]
    tall = tok_ref[...]
    n = emb.shape[0]

    sq = jnp.sum(emb * emb, axis=1, keepdims=True)
    inv = 1.0 / jnp.maximum(jnp.sqrt(sq), 1e-12)

    rows = emb_ref[pl.ds(i * r, r), :]
    rsq = jnp.sum(rows * rows, axis=1, keepdims=True)
    inv_rows = 1.0 / jnp.maximum(jnp.sqrt(rsq), 1e-12)
    trow = tok_ref[:, pl.ds(i * r, r)]

    sim = jnp.dot(rows, emb.T, preferred_element_type=jnp.float32)
    sim = sim * inv_rows * inv.T

    same = trow.T == tall
    col = jax.lax.broadcasted_iota(jnp.int32, (r, n), 1)
    row = jax.lax.broadcasted_iota(jnp.int32, (r, n), 0) + i * r
    valid = same & (col != row)

    e = jnp.where(valid, jnp.exp(sim * (-1.0 / _TEMPERATURE)), 0.0)
    s = jnp.sum(e, axis=1)
    cnt = jnp.sum(same.astype(jnp.int32), axis=1)
    in_group = cnt >= 2
    row_losses = jnp.where(in_group, -jnp.log(s), 0.0)

    loss_ref[...] = jnp.sum(row_losses).reshape(1, 1, 1)
    pairs_ref[...] = jnp.sum(jnp.where(in_group, cnt - 1, 0)).reshape(1, 1, 1)


def kernel(semantic_state, token_ids):
    b, t, d = semantic_state.shape
    n = b * t
    emb = semantic_state.reshape(n, d)
    toks = token_ids.reshape(1, n).astype(jnp.int32)
    grid = n // _ROW_TILE

    loss_parts, pair_parts = pl.pallas_call(
        _loss_tile_kernel,
        grid=(grid,),
        in_specs=[
            pl.BlockSpec((n, d), lambda i: (0, 0)),
            pl.BlockSpec((1, n), lambda i: (0, 0)),
        ],
        out_specs=[
            pl.BlockSpec((1, 1, 1), lambda i: (i, 0, 0)),
            pl.BlockSpec((1, 1, 1), lambda i: (i, 0, 0)),
        ],
        out_shape=[
            jax.ShapeDtypeStruct((grid, 1, 1), jnp.float32),
            jax.ShapeDtypeStruct((grid, 1, 1), jnp.int32),
        ],
        compiler_params=pltpu.CompilerParams(
            dimension_semantics=("parallel",),
        ),
    )(emb, toks)

    total_loss = jnp.sum(loss_parts)
    num_pairs = jnp.sum(pair_parts)
    result = jnp.where(
        num_pairs > 0,
        total_loss / jnp.maximum(num_pairs, 1).astype(jnp.float32),
        jnp.float32(0.0),
    )
    return result.astype(jnp.float32)


# speedup vs baseline: 1.9808x; 1.9808x over previous
import jax
import jax.numpy as jnp
from jax.experimental import pallas as pl
from jax.experimental.pallas import tpu as pltpu

_TEMPERATURE = 0.07
_R = 256
_C = 256


def _banded_kernel(emb_ref, tok_ref, loss_ref, pairs_ref):
    i = pl.program_id(0)
    n = emb_ref.shape[0]
    tall = tok_ref[...]

    rows = emb_ref[pl.ds(i * _R, _R), :]
    rsq = jnp.sum(rows * rows, axis=1, keepdims=True)
    inv_rows = 1.0 / jnp.maximum(jnp.sqrt(rsq), 1e-12)
    trow = tok_ref[:, pl.ds(i * _R, _R)]

    iota = jax.lax.broadcasted_iota(jnp.int32, (1, n), 1)
    t_first = jnp.max(jnp.where(iota == i * _R, tall, -1))
    t_last = jnp.max(jnp.where(iota == i * _R + _R - 1, tall, -1))
    jstart = jnp.min(jnp.where(tall == t_first, iota, n))
    jend = jnp.max(jnp.where(tall == t_last, iota, -1)) + 1
    c0 = jstart // _C
    c1 = (jend + _C - 1) // _C

    row_ids = jax.lax.broadcasted_iota(jnp.int32, (_R, _C), 0) + i * _R
    col_iota = jax.lax.broadcasted_iota(jnp.int32, (_R, _C), 1)

    def body(c, carry):
        s_acc, cnt_acc = carry
        cols = emb_ref[pl.ds(c * _C, _C), :]
        csq = jnp.sum(cols * cols, axis=1, keepdims=True)
        inv_cols = 1.0 / jnp.maximum(jnp.sqrt(csq), 1e-12)
        tcol = tok_ref[:, pl.ds(c * _C, _C)]
        sim = jax.lax.dot_general(
            rows, cols, (((1,), (1,)), ((), ())),
            preferred_element_type=jnp.float32)
        sim = sim * inv_rows * inv_cols.T
        same = trow.T == tcol
        valid = same & (col_iota + c * _C != row_ids)
        e = jnp.where(valid, jnp.exp(sim * (-1.0 / _TEMPERATURE)), 0.0)
        s_acc = s_acc + jnp.sum(e, axis=1, keepdims=True)
        cnt_acc = cnt_acc + jnp.sum(same.astype(jnp.int32), axis=1,
                                    keepdims=True)
        return s_acc, cnt_acc

    s0 = jnp.zeros((_R, 1), jnp.float32)
    n0 = jnp.zeros((_R, 1), jnp.int32)
    s, cnt = jax.lax.fori_loop(c0, c1, body, (s0, n0))

    in_group = cnt >= 2
    row_losses = jnp.where(in_group, -jnp.log(s), 0.0)
    loss_ref[...] = jnp.sum(row_losses).reshape(1, 1, 1)
    pairs_ref[...] = jnp.sum(jnp.where(in_group, cnt - 1, 0)).reshape(1, 1, 1)


def kernel(semantic_state, token_ids):
    b, t, d = semantic_state.shape
    n = b * t
    emb = semantic_state.reshape(n, d)
    toks = token_ids.reshape(n).astype(jnp.int32)

    order = jnp.argsort(toks)
    toks_s = toks[order].reshape(1, n)
    emb_s = emb[order]

    grid = n // _R
    loss_parts, pair_parts = pl.pallas_call(
        _banded_kernel,
        grid=(grid,),
        in_specs=[
            pl.BlockSpec((n, d), lambda i: (0, 0)),
            pl.BlockSpec((1, n), lambda i: (0, 0)),
        ],
        out_specs=[
            pl.BlockSpec((1, 1, 1), lambda i: (i, 0, 0)),
            pl.BlockSpec((1, 1, 1), lambda i: (i, 0, 0)),
        ],
        out_shape=[
            jax.ShapeDtypeStruct((grid, 1, 1), jnp.float32),
            jax.ShapeDtypeStruct((grid, 1, 1), jnp.int32),
        ],
        compiler_params=pltpu.CompilerParams(
            dimension_semantics=("parallel",),
        ),
    )(emb_s, toks_s)

    total_loss = jnp.sum(loss_parts)
    num_pairs = jnp.sum(pair_parts)
    result = jnp.where(
        num_pairs > 0,
        total_loss / jnp.maximum(num_pairs, 1).astype(jnp.float32),
        jnp.float32(0.0),
    )
    return result.astype(jnp.float32)


# speedup vs baseline: 2.1792x; 1.1001x over previous
import jax
import jax.numpy as jnp
from jax.experimental import pallas as pl
from jax.experimental.pallas import tpu as pltpu

_TEMPERATURE = 0.07
_R = 256
_C = 384


def _banded_kernel(emb_ref, tok_ref, loss_ref, pairs_ref):
    i = pl.program_id(0)
    npad = emb_ref.shape[0]
    tall = tok_ref[...]

    rows = emb_ref[pl.ds(i * _R, _R), :]
    rsq = jnp.sum(rows * rows, axis=1, keepdims=True)
    inv_rows = 1.0 / jnp.maximum(jnp.sqrt(rsq), 1e-12)
    trow = tok_ref[:, pl.ds(i * _R, _R)]

    iota = jax.lax.broadcasted_iota(jnp.int32, (1, npad), 1)
    t_first = jnp.max(jnp.where(iota == i * _R, tall, -1))
    t_last = jnp.max(jnp.where(iota == i * _R + _R - 1, tall, -1))
    jstart = jnp.min(jnp.where(tall == t_first, iota, npad))
    jend = jnp.max(jnp.where(tall == t_last, iota, -1)) + 1
    j0 = (jstart // 128) * 128
    n_iter = (jend - j0 + _C - 1) // _C

    col_iota = jax.lax.broadcasted_iota(jnp.int32, (_R, _C), 1)
    row_ids = jax.lax.broadcasted_iota(jnp.int32, (_R, _C), 0) + i * _R

    def body(k, carry):
        s_acc, cnt_acc = carry
        c = j0 + k * _C
        cols = emb_ref[pl.ds(c, _C), :]
        csq = jnp.sum(cols * cols, axis=1, keepdims=True)
        inv_cols = 1.0 / jnp.maximum(jnp.sqrt(csq), 1e-12)
        tcol = tok_ref[:, pl.ds(c, _C)]
        sim = jax.lax.dot_general(
            rows, cols, (((1,), (1,)), ((), ())),
            preferred_element_type=jnp.float32)
        sim = sim * inv_rows * inv_cols.T
        same = trow.T == tcol
        valid = same & (col_iota + c != row_ids)
        e = jnp.where(valid, jnp.exp(sim * (-1.0 / _TEMPERATURE)), 0.0)
        s_acc = s_acc + jnp.sum(e, axis=1, keepdims=True)
        cnt_acc = cnt_acc + jnp.sum(same.astype(jnp.int32), axis=1,
                                    keepdims=True)
        return s_acc, cnt_acc

    s0 = jnp.zeros((_R, 1), jnp.float32)
    n0 = jnp.zeros((_R, 1), jnp.int32)
    s, cnt = jax.lax.fori_loop(0, n_iter, body, (s0, n0))

    in_group = cnt >= 2
    row_losses = jnp.where(in_group, -jnp.log(s), 0.0)
    loss_ref[...] = jnp.sum(row_losses).reshape(1, 1, 1)
    pairs_ref[...] = jnp.sum(jnp.where(in_group, cnt - 1, 0)).reshape(1, 1, 1)


def kernel(semantic_state, token_ids):
    b, t, d = semantic_state.shape
    n = b * t
    emb = semantic_state.reshape(n, d)
    toks = token_ids.reshape(n).astype(jnp.int32)

    order = jnp.argsort(toks)
    toks_s = toks[order]
    emb_s = emb[order]
    toks_p = jnp.concatenate(
        [toks_s, jnp.full((_C,), -1, jnp.int32)]).reshape(1, n + _C)
    emb_p = jnp.concatenate([emb_s, jnp.ones((_C, d), jnp.float32)])

    grid = n // _R
    loss_parts, pair_parts = pl.pallas_call(
        _banded_kernel,
        grid=(grid,),
        in_specs=[
            pl.BlockSpec((n + _C, d), lambda i: (0, 0)),
            pl.BlockSpec((1, n + _C), lambda i: (0, 0)),
        ],
        out_specs=[
            pl.BlockSpec((1, 1, 1), lambda i: (i, 0, 0)),
            pl.BlockSpec((1, 1, 1), lambda i: (i, 0, 0)),
        ],
        out_shape=[
            jax.ShapeDtypeStruct((grid, 1, 1), jnp.float32),
            jax.ShapeDtypeStruct((grid, 1, 1), jnp.int32),
        ],
        compiler_params=pltpu.CompilerParams(
            dimension_semantics=("parallel",),
        ),
    )(emb_p, toks_p)

    total_loss = jnp.sum(loss_parts)
    num_pairs = jnp.sum(pair_parts)
    result = jnp.where(
        num_pairs > 0,
        total_loss / jnp.maximum(num_pairs, 1).astype(jnp.float32),
        jnp.float32(0.0),
    )
    return result.astype(jnp.float32)


# speedup vs baseline: 2.1983x; 1.0088x over previous
import jax
import jax.numpy as jnp
from jax.experimental import pallas as pl
from jax.experimental.pallas import tpu as pltpu

_TEMPERATURE = 0.07
_R = 256
_C = 384
_PAD_TOKEN = 1 << 30


def _lower_bound(tok_sm, target, npad):
    def body(_, lo_hi):
        lo, hi = lo_hi
        mid = (lo + hi) // 2
        go_right = (lo < hi) & (tok_sm[mid] < target)
        shrink = (lo < hi) & jnp.logical_not(go_right)
        return (jnp.where(go_right, mid + 1, lo),
                jnp.where(shrink, mid, hi))
    lo, _ = jax.lax.fori_loop(0, 14, body, (jnp.int32(0), jnp.int32(npad)))
    return lo


def _banded_kernel(tok_sm, emb_ref, tok_ref, loss_ref, pairs_ref):
    i = pl.program_id(0)

    rows = emb_ref[pl.ds(i * _R, _R), :]
    rsq = jnp.sum(rows * rows, axis=1, keepdims=True)
    inv_rows = jax.lax.rsqrt(jnp.maximum(rsq, 1e-24))
    trow = tok_ref[:, pl.ds(i * _R, _R)]

    npad = tok_sm.shape[0]
    t_first = tok_sm[i * _R]
    t_last = tok_sm[i * _R + _R - 1]
    jstart = _lower_bound(tok_sm, t_first, npad)
    jend = _lower_bound(tok_sm, t_last + 1, npad)
    j0 = (jstart // 128) * 128
    n_iter = (jend - j0 + _C - 1) // _C

    def body(k, carry):
        s_acc, cnt_acc = carry
        c = j0 + k * _C
        cols = emb_ref[pl.ds(c, _C), :]
        csq = jnp.sum(cols * cols, axis=1, keepdims=True)
        inv_cols = jax.lax.rsqrt(jnp.maximum(csq, 1e-24))
        tcol = tok_ref[:, pl.ds(c, _C)]
        sim = jax.lax.dot_general(
            rows, cols, (((1,), (1,)), ((), ())),
            preferred_element_type=jnp.float32)
        sim = sim * inv_rows * inv_cols.T
        same = trow.T == tcol
        e = jnp.where(same, jnp.exp(sim * (-1.0 / _TEMPERATURE)), 0.0)
        s_acc = s_acc + jnp.sum(e, axis=1, keepdims=True)
        cnt_acc = cnt_acc + jnp.sum(same.astype(jnp.int32), axis=1,
                                    keepdims=True)
        return s_acc, cnt_acc

    s0 = jnp.zeros((_R, 1), jnp.float32)
    n0 = jnp.zeros((_R, 1), jnp.int32)
    s, cnt = jax.lax.fori_loop(0, n_iter, body, (s0, n0))

    self_sim = rsq * inv_rows * inv_rows
    s = s - jnp.exp(self_sim * (-1.0 / _TEMPERATURE))

    in_group = cnt >= 2
    row_losses = jnp.where(in_group, -jnp.log(s), 0.0)
    loss_ref[...] = jnp.sum(row_losses).reshape(1, 1, 1)
    pairs_ref[...] = jnp.sum(jnp.where(in_group, cnt - 1, 0)).reshape(1, 1, 1)


def kernel(semantic_state, token_ids):
    b, t, d = semantic_state.shape
    n = b * t
    emb = semantic_state.reshape(n, d)
    toks = token_ids.reshape(n).astype(jnp.int32)

    order = jnp.argsort(toks)
    toks_s = toks[order]
    emb_s = emb[order]
    toks_p = jnp.concatenate([toks_s, jnp.full((_C,), _PAD_TOKEN, jnp.int32)])
    emb_p = jnp.concatenate([emb_s, jnp.ones((_C, d), jnp.float32)])
    toks_row = toks_p.reshape(1, n + _C)

    grid = n // _R
    grid_spec = pltpu.PrefetchScalarGridSpec(
        num_scalar_prefetch=1,
        grid=(grid,),
        in_specs=[
            pl.BlockSpec((n + _C, d), lambda i, sm: (0, 0)),
            pl.BlockSpec((1, n + _C), lambda i, sm: (0, 0)),
        ],
        out_specs=[
            pl.BlockSpec((1, 1, 1), lambda i, sm: (i, 0, 0)),
            pl.BlockSpec((1, 1, 1), lambda i, sm: (i, 0, 0)),
        ],
    )
    loss_parts, pair_parts = pl.pallas_call(
        _banded_kernel,
        grid_spec=grid_spec,
        out_shape=[
            jax.ShapeDtypeStruct((grid, 1, 1), jnp.float32),
            jax.ShapeDtypeStruct((grid, 1, 1), jnp.int32),
        ],
        compiler_params=pltpu.CompilerParams(
            dimension_semantics=("parallel",),
        ),
    )(toks_p, emb_p, toks_row)

    total_loss = jnp.sum(loss_parts)
    num_pairs = jnp.sum(pair_parts)
    result = jnp.where(
        num_pairs > 0,
        total_loss / jnp.maximum(num_pairs, 1).astype(jnp.float32),
        jnp.float32(0.0),
    )
    return result.astype(jnp.float32)


# speedup vs baseline: 2.3741x; 1.0800x over previous
import jax
import jax.numpy as jnp
from jax.experimental import pallas as pl
from jax.experimental.pallas import tpu as pltpu

_TEMPERATURE = 0.07
_R = 256
_C = 384
_PAD_TOKEN = 1 << 30


def _lower_bound(tok_sm, target, npad):
    def body(_, lo_hi):
        lo, hi = lo_hi
        mid = (lo + hi) // 2
        go_right = (lo < hi) & (tok_sm[mid] < target)
        shrink = (lo < hi) & jnp.logical_not(go_right)
        return (jnp.where(go_right, mid + 1, lo),
                jnp.where(shrink, mid, hi))
    lo, _ = jax.lax.fori_loop(0, 14, body, (jnp.int32(0), jnp.int32(npad)))
    return lo


def _banded_kernel(tok_sm, emb_ref, tok_ref, loss_ref, pairs_ref, nemb_ref):
    npad = emb_ref.shape[0]
    n = npad - _C
    n_tiles = n // _R
    scale = jnp.float32(1.0 / _TEMPERATURE) ** 0.5

    def norm_body(k, _):
        x = emb_ref[pl.ds(k * _R, _R), :]
        sq = jnp.sum(x * x, axis=1, keepdims=True)
        nemb_ref[pl.ds(k * _R, _R), :] = x * (
            jax.lax.rsqrt(jnp.maximum(sq, 1e-24)) * scale)
        return 0
    jax.lax.fori_loop(0, npad // _R, norm_body, 0)

    tall = tok_ref
    ones_c = jnp.ones((_C, 128), jnp.float32)

    def tile_body(i, carry):
        loss_acc, pairs_acc = carry
        rows = nemb_ref[pl.ds(i * _R, _R), :]
        trow = tall[:, pl.ds(i * _R, _R)]

        t_first = tok_sm[i * _R]
        t_last = tok_sm[i * _R + _R - 1]
        jstart = _lower_bound(tok_sm, t_first, npad)
        jend = _lower_bound(tok_sm, t_last + 1, npad)
        j0 = (jstart // 128) * 128
        n_iter = (jend - j0 + _C - 1) // _C

        def body(k, acc):
            s_acc, cnt_acc = acc
            c = j0 + k * _C
            cols = nemb_ref[pl.ds(c, _C), :]
            tcol = tall[:, pl.ds(c, _C)]
            simt = jax.lax.dot_general(
                rows, cols, (((1,), (1,)), ((), ())),
                preferred_element_type=jnp.float32)
            same = trow.T == tcol
            e = jnp.where(same, jnp.exp(-simt), 0.0)
            samef = jnp.where(same, 1.0, 0.0)
            s_part = jax.lax.dot_general(
                e, ones_c, (((1,), (0,)), ((), ())),
                preferred_element_type=jnp.float32)[:, :1]
            c_part = jax.lax.dot_general(
                samef, ones_c, (((1,), (0,)), ((), ())),
                preferred_element_type=jnp.float32)[:, :1]
            return s_acc + s_part, cnt_acc + c_part

        s0 = jnp.zeros((_R, 1), jnp.float32)
        c0 = jnp.zeros((_R, 1), jnp.float32)
        s, cntf = jax.lax.fori_loop(0, n_iter, body, (s0, c0))

        ssq = jnp.sum(rows * rows, axis=1, keepdims=True)
        s = s - jnp.exp(-ssq)

        in_group = cntf >= 2.0
        loss_acc = loss_acc + jnp.where(in_group, -jnp.log(s), 0.0)
        pairs_acc = pairs_acc + jnp.where(in_group, cntf - 1.0, 0.0)
        return loss_acc, pairs_acc

    z = jnp.zeros((_R, 1), jnp.float32)
    loss_v, pairs_v = jax.lax.fori_loop(0, n_tiles, tile_body, (z, z))
    loss_ref[...] = jnp.sum(loss_v).reshape(1, 1)
    pairs_ref[...] = jnp.sum(pairs_v.astype(jnp.int32)).reshape(1, 1)


def kernel(semantic_state, token_ids):
    b, t, d = semantic_state.shape
    n = b * t
    emb = semantic_state.reshape(n, d)
    toks = token_ids.reshape(n).astype(jnp.int32)

    order = jnp.argsort(toks)
    toks_s = toks[order]
    emb_s = emb[order]
    toks_p = jnp.concatenate([toks_s, jnp.full((_C,), _PAD_TOKEN, jnp.int32)])
    emb_p = jnp.concatenate([emb_s, jnp.ones((_C, d), jnp.float32)])
    toks_row = toks_p.reshape(1, n + _C)

    grid_spec = pltpu.PrefetchScalarGridSpec(
        num_scalar_prefetch=1,
        grid=(1,),
        in_specs=[
            pl.BlockSpec((n + _C, d), lambda i, sm: (0, 0)),
            pl.BlockSpec((1, n + _C), lambda i, sm: (0, 0)),
        ],
        out_specs=[
            pl.BlockSpec((1, 1), lambda i, sm: (0, 0)),
            pl.BlockSpec((1, 1), lambda i, sm: (0, 0)),
        ],
        scratch_shapes=[pltpu.VMEM((n + _C, d), jnp.float32)],
    )
    loss, pairs = pl.pallas_call(
        _banded_kernel,
        grid_spec=grid_spec,
        out_shape=[
            jax.ShapeDtypeStruct((1, 1), jnp.float32),
            jax.ShapeDtypeStruct((1, 1), jnp.int32),
        ],
    )(toks_p, emb_p, toks_row)

    total_loss = loss[0, 0]
    num_pairs = pairs[0, 0]
    result = jnp.where(
        num_pairs > 0,
        total_loss / jnp.maximum(num_pairs, 1).astype(jnp.float32),
        jnp.float32(0.0),
    )
    return result.astype(jnp.float32)


# speedup vs baseline: 3.4153x; 1.4386x over previous
import functools

import jax
import jax.numpy as jnp
from jax import lax
from jax.experimental import pallas as pl
from jax.experimental.pallas import tpu as pltpu
from jax.experimental.pallas import tpu_sc as plsc

_TEMPERATURE = 0.07
_R = 512
_C = 768
_PAD_TOKEN = 1 << 30


def _lower_bound(tok_sm, target, npad):
    def body(_, lo_hi):
        lo, hi = lo_hi
        mid = (lo + hi) // 2
        go_right = (lo < hi) & (tok_sm[mid] < target)
        shrink = (lo < hi) & jnp.logical_not(go_right)
        return (jnp.where(go_right, mid + 1, lo),
                jnp.where(shrink, mid, hi))
    lo, _ = jax.lax.fori_loop(0, 14, body, (jnp.int32(0), jnp.int32(npad)))
    return lo


def _banded_kernel(tok_sm, emb_ref, tok_ref, loss_ref, pairs_ref, nemb_ref):
    npad = emb_ref.shape[0]
    n = npad - _C
    n_tiles = n // _R
    scale = jnp.float32(1.0 / _TEMPERATURE) ** 0.5

    def norm_body(k, _):
        x = emb_ref[pl.ds(k * _R, _R), :]
        sq = jnp.sum(x * x, axis=1, keepdims=True)
        nemb_ref[pl.ds(k * _R, _R), :] = x * (
            jax.lax.rsqrt(jnp.maximum(sq, 1e-24)) * scale)
        return 0
    jax.lax.fori_loop(0, npad // _R, norm_body, 0)

    tall = tok_ref
    ones_c = jnp.ones((_C, 128), jnp.float32)

    loss_acc = jnp.zeros((_R, 1), jnp.float32)
    pairs_acc = jnp.zeros((_R, 1), jnp.float32)

    for i in range(n_tiles):
        rows = nemb_ref[pl.ds(i * _R, _R), :]
        trow = tall[:, pl.ds(i * _R, _R)]

        t_first = tok_sm[i * _R]
        t_last = tok_sm[i * _R + _R - 1]
        jstart = _lower_bound(tok_sm, t_first, npad)
        jend = _lower_bound(tok_sm, t_last + 1, npad)
        j0 = (jstart // 128) * 128
        n_iter = (jend - j0 + _C - 1) // _C

        def window(c, rows=rows, trow=trow):
            cols = nemb_ref[pl.ds(c, _C), :]
            tcol = tall[:, pl.ds(c, _C)]
            simt = jax.lax.dot_general(
                rows, cols, (((1,), (1,)), ((), ())),
                preferred_element_type=jnp.float32)
            same = trow.T == tcol
            e = jnp.where(same, jnp.exp(-simt), 0.0)
            samef = jnp.where(same, 1.0, 0.0)
            s_part = jax.lax.dot_general(
                e, ones_c, (((1,), (0,)), ((), ())),
                preferred_element_type=jnp.float32)[:, :1]
            c_part = jax.lax.dot_general(
                samef, ones_c, (((1,), (0,)), ((), ())),
                preferred_element_type=jnp.float32)[:, :1]
            return s_part, c_part

        s, cntf = window(j0)

        def extra(k, acc):
            s_acc, cnt_acc = acc
            s_part, c_part = window(j0 + k * _C)
            return s_acc + s_part, cnt_acc + c_part

        s, cntf = jax.lax.fori_loop(1, n_iter, extra, (s, cntf))

        ssq = jnp.sum(rows * rows, axis=1, keepdims=True)
        s = s - jnp.exp(-ssq)

        in_group = cntf >= 2.0
        loss_acc = loss_acc + jnp.where(in_group, -jnp.log(s), 0.0)
        pairs_acc = pairs_acc + jnp.where(in_group, cntf - 1.0, 0.0)

    loss_ref[...] = jnp.sum(loss_acc).reshape(1, 1)
    pairs_ref[...] = jnp.sum(pairs_acc.astype(jnp.int32)).reshape(1, 1)


def _make_sc_gather(n, d):
    info = plsc.get_sparse_core_info()
    nc, ns = info.num_cores, info.num_subcores
    nw = nc * ns
    assert n % (8 * nw) == 0
    b_per_w = n // nw
    mesh = plsc.VectorSubcoreMesh(core_axis_name="c", subcore_axis_name="s")

    @functools.partial(
        pl.kernel, mesh=mesh,
        out_type=jax.ShapeDtypeStruct((n, d), jnp.float32),
        compiler_params=pltpu.CompilerParams(use_tc_tiling_on_sc=False),
        scratch_types=[
            pltpu.VMEM((b_per_w,), jnp.int32),
            pltpu.VMEM((b_per_w, d), jnp.float32),
            pltpu.SemaphoreType.DMA,
        ],
    )
    def gather_rows(table_hbm, idx_hbm, out_hbm, idx_v, rows_v, sem):
        wid = lax.axis_index("s") * nc + lax.axis_index("c")
        base = wid * b_per_w
        pltpu.sync_copy(idx_hbm.at[pl.ds(base, b_per_w)], idx_v)
        pltpu.async_copy(table_hbm.at[idx_v], rows_v, sem).wait()
        pltpu.sync_copy(rows_v, out_hbm.at[pl.ds(base, b_per_w)])

    return gather_rows


def kernel(semantic_state, token_ids):
    b, t, d = semantic_state.shape
    n = b * t
    emb = semantic_state.reshape(n, d)
    toks = token_ids.reshape(n).astype(jnp.int32)

    toks_s, order = jax.lax.sort(
        (toks, jax.lax.iota(jnp.int32, n)), num_keys=1)
    emb_s = _make_sc_gather(n, d)(emb, order)
    toks_p = jnp.concatenate([toks_s, jnp.full((_C,), _PAD_TOKEN, jnp.int32)])
    emb_p = jnp.concatenate([emb_s, jnp.ones((_C, d), jnp.float32)])
    toks_row = toks_p.reshape(1, n + _C)

    grid_spec = pltpu.PrefetchScalarGridSpec(
        num_scalar_prefetch=1,
        grid=(1,),
        in_specs=[
            pl.BlockSpec((n + _C, d), lambda i, sm: (0, 0)),
            pl.BlockSpec((1, n + _C), lambda i, sm: (0, 0)),
        ],
        out_specs=[
            pl.BlockSpec((1, 1), lambda i, sm: (0, 0)),
            pl.BlockSpec((1, 1), lambda i, sm: (0, 0)),
        ],
        scratch_shapes=[pltpu.VMEM((n + _C, d), jnp.float32)],
    )
    loss, pairs = pl.pallas_call(
        _banded_kernel,
        grid_spec=grid_spec,
        out_shape=[
            jax.ShapeDtypeStruct((1, 1), jnp.float32),
            jax.ShapeDtypeStruct((1, 1), jnp.int32),
        ],
    )(toks_p, emb_p, toks_row)

    total_loss = loss[0, 0]
    num_pairs = pairs[0, 0]
    result = jnp.where(
        num_pairs > 0,
        total_loss / jnp.maximum(num_pairs, 1).astype(jnp.float32),
        jnp.float32(0.0),
    )
    return result.astype(jnp.float32)
